# Initial kernel scaffold; baseline (speedup 1.0000x reference)
import functools

import jax
import jax.numpy as jnp
from jax.experimental import pallas as pl
from jax.experimental.pallas import tpu as pltpu

N1 = 320000
D = 128
BM = 1024


def _epilogue_body(s1_ref, s2_ref, x_ref, w_a_ref, w_b_ref, w_skip_ref,
                   g_ref, b_ref, out_ref):
    acc = jnp.dot(s1_ref[...], w_a_ref[...].T, preferred_element_type=jnp.float32)
    acc += jnp.dot(s2_ref[...], w_b_ref[...].T, preferred_element_type=jnp.float32)
    acc += jnp.dot(x_ref[...], w_skip_ref[...].T, preferred_element_type=jnp.float32)
    m = jnp.mean(acc, axis=-1, keepdims=True)
    c = acc - m
    v = jnp.mean(c * c, axis=-1, keepdims=True)
    y = c * jax.lax.rsqrt(v + 1e-5) * g_ref[...] + b_ref[...]
    out_ref[...] = jax.nn.gelu(y, approximate=False)


def _epilogue(s1, s2, x, w_a, w_b, w_skip, g, b):
    grid = (N1 // BM,)
    blk = pl.BlockSpec((BM, D), lambda i: (i, 0))
    wblk = pl.BlockSpec((D, D), lambda i: (0, 0))
    vblk = pl.BlockSpec((1, D), lambda i: (0, 0))
    return pl.pallas_call(
        _epilogue_body,
        grid=grid,
        in_specs=[blk, blk, blk, wblk, wblk, wblk, vblk, vblk],
        out_specs=blk,
        out_shape=jax.ShapeDtypeStruct((N1, D), jnp.float32),
    )(s1, s2, x, w_a, w_b, w_skip, g.reshape(1, D), b.reshape(1, D))


def _spmm(rows, cols, vals, X):
    gathered = vals[:, None] * jnp.take(X, cols, axis=0)
    return jax.ops.segment_sum(gathered, rows, num_segments=N1)


def kernel(X_equ, X_inv, W1, W2, W3, W4, W5, W6, g_e, b_e, g_i, b_i,
           vals_Le, vals_ie, vals_Li, vals_ei,
           rows_Le, cols_Le, rows_ie, cols_ie, rows_Li, cols_Li,
           rows_ei, cols_ei):
    s_Le = _spmm(rows_Le, cols_Le, vals_Le, X_equ)
    s_ie = _spmm(rows_ie, cols_ie, vals_ie, X_inv)
    s_Li = _spmm(rows_Li, cols_Li, vals_Li, X_inv)
    s_ei = _spmm(rows_ei, cols_ei, vals_ei, X_equ)
    out_equ = _epilogue(s_Le, s_ie, X_equ, W1, W2, W5, g_e, b_e)
    out_inv = _epilogue(s_Li, s_ei, X_inv, W3, W4, W6, g_i, b_i)
    return (out_equ, out_inv)


# speedup vs baseline: 1.0391x; 1.0391x over previous
import functools

import jax
import jax.numpy as jnp
from jax.experimental import pallas as pl
from jax.experimental.pallas import tpu as pltpu

N1 = 320000
D = 128
BM = 2560


def _epilogue_body(s1_ref, s2_ref, x_ref, w_a_ref, w_b_ref, w_skip_ref,
                   g_ref, b_ref, out_ref):
    acc = jnp.dot(s1_ref[...], w_a_ref[...].T, preferred_element_type=jnp.float32)
    acc += jnp.dot(s2_ref[...], w_b_ref[...].T, preferred_element_type=jnp.float32)
    acc += jnp.dot(x_ref[...], w_skip_ref[...].T, preferred_element_type=jnp.float32)
    m = jnp.mean(acc, axis=-1, keepdims=True)
    c = acc - m
    v = jnp.mean(c * c, axis=-1, keepdims=True)
    y = c * jax.lax.rsqrt(v + 1e-5) * g_ref[...] + b_ref[...]
    out_ref[...] = 0.5 * y * (1.0 + jax.lax.erf(y * 0.7071067811865476))


def _epilogue(s1, s2, x, w_a, w_b, w_skip, g, b):
    grid = (N1 // BM,)
    blk = pl.BlockSpec((BM, D), lambda i: (i, 0))
    wblk = pl.BlockSpec((D, D), lambda i: (0, 0))
    vblk = pl.BlockSpec((1, D), lambda i: (0, 0))
    return pl.pallas_call(
        _epilogue_body,
        grid=grid,
        in_specs=[blk, blk, blk, wblk, wblk, wblk, vblk, vblk],
        out_specs=blk,
        out_shape=jax.ShapeDtypeStruct((N1, D), jnp.float32),
    )(s1, s2, x, w_a, w_b, w_skip, g.reshape(1, D), b.reshape(1, D))


def _spmm(rows, cols, vals, X):
    gathered = vals[:, None] * jnp.take(X, cols, axis=0)
    return jax.ops.segment_sum(gathered, rows, num_segments=N1)


def kernel(X_equ, X_inv, W1, W2, W3, W4, W5, W6, g_e, b_e, g_i, b_i,
           vals_Le, vals_ie, vals_Li, vals_ei,
           rows_Le, cols_Le, rows_ie, cols_ie, rows_Li, cols_Li,
           rows_ei, cols_ei):
    s_Le = _spmm(rows_Le, cols_Le, vals_Le, X_equ)
    s_ie = _spmm(rows_ie, cols_ie, vals_ie, X_inv)
    s_Li = _spmm(rows_Li, cols_Li, vals_Li, X_inv)
    s_ei = _spmm(rows_ei, cols_ei, vals_ei, X_equ)
    out_equ = _epilogue(s_Le, s_ie, X_equ, W1, W2, W5, g_e, b_e)
    out_inv = _epilogue(s_Li, s_ei, X_inv, W3, W4, W6, g_i, b_i)
    return (out_equ, out_inv)
